# baseline (speedup 1.0000x reference)
import jax
import jax.numpy as jnp
from jax import lax
from jax.experimental import pallas as pl
from jax.experimental.pallas import tpu as pltpu
from jax.experimental.pallas import tpu_sc as plsc

N_NODES = 50000
N_EDGES = 800000
N_EMB = 30
N_DIST = 100
N_HID = 60
DP = 32

NC, NS = 2, 16
NW = NC * NS

G_ROWS = 6400
G_ROWS_W = G_ROWS // NW
G_CHUNK = 25
G_NCHUNK = G_ROWS_W // G_CHUNK

EB = 6400
GRID = N_EDGES // EB
WIN = 256
ACC_L = 50176


def _sc_mesh():
    return plsc.VectorSubcoreMesh(core_axis_name="c", subcore_axis_name="s",
                                  num_cores=NC, num_subcores=NS)


def _gather_body(table, idx2, out, idx_v, rows_v, sem):
    c = lax.axis_index("c")
    s = lax.axis_index("s")
    w = s * NC + c
    row0 = w * G_ROWS_W

    def chunk(t, carry):
        r0 = row0 + t * G_CHUNK
        pltpu.sync_copy(idx2.at[pl.ds(r0, G_CHUNK)], idx_v)
        descs = [
            pltpu.async_copy(table.at[idx_v.at[j]],
                             rows_v.at[pl.ds(j * 128, 128)], sem)
            for j in range(G_CHUNK)
        ]
        for d in descs:
            d.wait()
        pltpu.sync_copy(rows_v, out.at[pl.ds(r0 * 128, G_CHUNK * 128)])
        return carry

    lax.fori_loop(0, G_NCHUNK, chunk, 0)


def _mega_body(dt_ref, g_ref, mi_ref, aft_ref, wdf_t, wcf_tp, wcf_t, wfc_t,
               bdf_t, bcf_t, out_ref, acc_ref):
    i = pl.program_id(0)
    f32 = jnp.float32

    @pl.when(i == 0)
    def _():
        acc_ref[...] = jnp.zeros((N_EMB, ACC_L), f32)

    bf16 = jnp.bfloat16
    dh_t = jnp.dot(wdf_t[...].astype(bf16), dt_ref[...].astype(bf16),
                   preferred_element_type=f32)
    dh_t = dh_t + bdf_t[...]
    g_t = jnp.swapaxes(g_ref[...], 0, 1)
    gh_t = jnp.dot(wcf_tp[...].astype(bf16), g_t,
                   preferred_element_type=f32) + bcf_t[...]
    p_t = dh_t * gh_t
    o_t = jnp.tanh(jnp.dot(wfc_t[...].astype(bf16), p_t.astype(bf16),
                           preferred_element_type=f32))

    ids = mi_ref[...].reshape(1, EB)
    lo = jnp.min(ids)
    hi = jnp.max(ids)
    base0 = (lo // 128) * 128
    nwin = (hi - base0) // WIN + 1

    def win(t, carry):
        wb = base0 + t * WIN
        iota = lax.broadcasted_iota(jnp.int32, (WIN, EB), 0) + wb
        oh = (iota == ids).astype(jnp.bfloat16)
        partial = lax.dot_general(o_t.astype(jnp.bfloat16), oh,
                                  (((1,), (1,)), ((), ())),
                                  preferred_element_type=f32)
        acc_ref[:, pl.ds(wb, WIN)] += partial
        return carry

    lax.fori_loop(0, nwin, win, 0)

    @pl.when(i == GRID - 1)
    def _():
        aft = aft_ref[...]
        afh_t = jnp.dot(wcf_t[...], aft, preferred_element_type=f32)
        afh_t = afh_t + bcf_t[...]
        self_t = jnp.tanh(jnp.dot(wfc_t[...], afh_t * bdf_t[...],
                                  preferred_element_type=f32))
        out_ref[...] = acc_ref[:, :N_NODES] + aft - self_t


def kernel(atom_features, distance, distance_membership_i,
           distance_membership_j, W_cf, W_df, W_fc, b_cf, b_df):
    f32 = jnp.float32
    mi3 = distance_membership_i.astype(jnp.int32).reshape(GRID, 1, EB)
    mj = distance_membership_j.astype(jnp.int32)
    mj2 = jnp.pad(mj, (0, G_ROWS * 128 - N_EDGES)).reshape(G_ROWS, 128)
    af_pad = jnp.pad(atom_features, ((0, 0), (0, DP - N_EMB)))
    dist_t = distance.T
    af_t = atom_features.T
    wdf_t = W_df.T
    wcf_tp = jnp.pad(W_cf, ((0, DP - N_EMB), (0, 0))).T
    wcf_t = W_cf.T
    wfc_t = W_fc.T
    bdf_t = b_df.reshape(N_HID, 1)
    bcf_t = b_cf.reshape(N_HID, 1)

    gather_call = pl.kernel(
        _gather_body,
        out_type=jax.ShapeDtypeStruct((G_ROWS * 128, DP), jnp.bfloat16),
        mesh=_sc_mesh(),
        scratch_types=[
            pltpu.VMEM((G_CHUNK, 128), jnp.int32),
            pltpu.VMEM((G_CHUNK * 128, DP), jnp.bfloat16),
            pltpu.SemaphoreType.DMA,
        ],
        compiler_params=pltpu.CompilerParams(use_tc_tiling_on_sc=False),
    )
    g0 = gather_call(af_pad.astype(jnp.bfloat16), mj2)

    out_t = pl.pallas_call(
        _mega_body,
        grid=(GRID,),
        in_specs=[
            pl.BlockSpec((N_DIST, EB), lambda i: (0, i)),
            pl.BlockSpec((EB, DP), lambda i: (i, 0)),
            pl.BlockSpec((1, 1, EB), lambda i: (i, 0, 0)),
            pl.BlockSpec((N_EMB, N_NODES), lambda i: (0, 0)),
            pl.BlockSpec((N_HID, N_DIST), lambda i: (0, 0)),
            pl.BlockSpec((N_HID, DP), lambda i: (0, 0)),
            pl.BlockSpec((N_HID, N_EMB), lambda i: (0, 0)),
            pl.BlockSpec((N_EMB, N_HID), lambda i: (0, 0)),
            pl.BlockSpec((N_HID, 1), lambda i: (0, 0)),
            pl.BlockSpec((N_HID, 1), lambda i: (0, 0)),
        ],
        out_specs=pl.BlockSpec((N_EMB, N_NODES), lambda i: (0, 0)),
        out_shape=jax.ShapeDtypeStruct((N_EMB, N_NODES), f32),
        scratch_shapes=[pltpu.VMEM((N_EMB, ACC_L), f32)],
    )(dist_t, g0, mi3, af_t, wdf_t, wcf_tp, wcf_t, wfc_t, bdf_t, bcf_t)

    return out_t.T


# speedup vs baseline: 1.0072x; 1.0072x over previous
import jax
import jax.numpy as jnp
from jax import lax
from jax.experimental import pallas as pl
from jax.experimental.pallas import tpu as pltpu
from jax.experimental.pallas import tpu_sc as plsc

N_NODES = 50000
N_EDGES = 800000
N_EMB = 30
N_DIST = 100
N_HID = 60
DP = 32

NC, NS = 2, 16
NW = NC * NS

G_ROWS = 6400
G_ROWS_W = G_ROWS // NW
G_CHUNK = 25
G_NCHUNK = G_ROWS_W // G_CHUNK

EB = 6400
GRID = N_EDGES // EB
WIN = 256
ACC_L = 50176


def _sc_mesh():
    return plsc.VectorSubcoreMesh(core_axis_name="c", subcore_axis_name="s",
                                  num_cores=NC, num_subcores=NS)


def _gather_body(table, idx2, out, idx_v, rows_v, sem):
    c = lax.axis_index("c")
    s = lax.axis_index("s")
    w = s * NC + c
    row0 = w * G_ROWS_W

    def chunk(t, carry):
        r0 = row0 + t * G_CHUNK
        pltpu.sync_copy(idx2.at[pl.ds(r0, G_CHUNK)], idx_v)
        descs = [
            pltpu.async_copy(table.at[idx_v.at[j]],
                             rows_v.at[pl.ds(j * 128, 128)], sem)
            for j in range(G_CHUNK)
        ]
        for d in descs:
            d.wait()
        pltpu.sync_copy(rows_v, out.at[pl.ds(r0 * 128, G_CHUNK * 128)])
        return carry

    lax.fori_loop(0, G_NCHUNK, chunk, 0)


def _mega_body(dt_ref, g_ref, mi_ref, aft_ref, wdf_t, wcf_tp, wcf_t, wfc_t,
               bdf_t, bcf_t, out_ref, acc_ref):
    i = pl.program_id(0)
    f32 = jnp.float32

    @pl.when(i == 0)
    def _():
        acc_ref[...] = jnp.zeros((N_EMB, ACC_L), f32)

    dh_t = jnp.dot(wdf_t[...], dt_ref[...], preferred_element_type=f32)
    dh_t = dh_t + bdf_t[...]
    g_t = jnp.swapaxes(g_ref[...], 0, 1).astype(f32)
    gh_t = jnp.dot(wcf_tp[...], g_t, preferred_element_type=f32) + bcf_t[...]
    p_t = dh_t * gh_t
    o_t = jnp.tanh(jnp.dot(wfc_t[...], p_t, preferred_element_type=f32))

    ids = mi_ref[...].reshape(1, EB)
    lo = jnp.min(ids)
    hi = jnp.max(ids)
    base0 = (lo // 128) * 128
    nwin = (hi - base0) // WIN + 1

    def win(t, carry):
        wb = base0 + t * WIN
        iota = lax.broadcasted_iota(jnp.int32, (WIN, EB), 0) + wb
        oh = (iota == ids).astype(f32)
        partial = lax.dot_general(o_t, oh, (((1,), (1,)), ((), ())),
                                  preferred_element_type=f32)
        acc_ref[:, pl.ds(wb, WIN)] += partial
        return carry

    lax.fori_loop(0, nwin, win, 0)

    @pl.when(i == GRID - 1)
    def _():
        aft = aft_ref[...]
        afh_t = jnp.dot(wcf_t[...], aft, preferred_element_type=f32)
        afh_t = afh_t + bcf_t[...]
        self_t = jnp.tanh(jnp.dot(wfc_t[...], afh_t * bdf_t[...],
                                  preferred_element_type=f32))
        out_ref[...] = acc_ref[:, :N_NODES] + aft - self_t


def kernel(atom_features, distance, distance_membership_i,
           distance_membership_j, W_cf, W_df, W_fc, b_cf, b_df):
    f32 = jnp.float32
    mi3 = distance_membership_i.astype(jnp.int32).reshape(GRID, 1, EB)
    mj = distance_membership_j.astype(jnp.int32)
    mj2 = jnp.pad(mj, (0, G_ROWS * 128 - N_EDGES)).reshape(G_ROWS, 128)
    af_pad = jnp.pad(atom_features, ((0, 0), (0, DP - N_EMB)))
    dist_t = distance.T
    af_t = atom_features.T
    wdf_t = W_df.T
    wcf_tp = jnp.pad(W_cf, ((0, DP - N_EMB), (0, 0))).T
    wcf_t = W_cf.T
    wfc_t = W_fc.T
    bdf_t = b_df.reshape(N_HID, 1)
    bcf_t = b_cf.reshape(N_HID, 1)

    gather_call = pl.kernel(
        _gather_body,
        out_type=jax.ShapeDtypeStruct((G_ROWS * 128, DP), jnp.bfloat16),
        mesh=_sc_mesh(),
        scratch_types=[
            pltpu.VMEM((G_CHUNK, 128), jnp.int32),
            pltpu.VMEM((G_CHUNK * 128, DP), jnp.bfloat16),
            pltpu.SemaphoreType.DMA,
        ],
        compiler_params=pltpu.CompilerParams(use_tc_tiling_on_sc=False),
    )
    g0 = gather_call(af_pad.astype(jnp.bfloat16), mj2)

    out_t = pl.pallas_call(
        _mega_body,
        grid=(GRID,),
        in_specs=[
            pl.BlockSpec((N_DIST, EB), lambda i: (0, i)),
            pl.BlockSpec((EB, DP), lambda i: (i, 0)),
            pl.BlockSpec((1, 1, EB), lambda i: (i, 0, 0)),
            pl.BlockSpec((N_EMB, N_NODES), lambda i: (0, 0)),
            pl.BlockSpec((N_HID, N_DIST), lambda i: (0, 0)),
            pl.BlockSpec((N_HID, DP), lambda i: (0, 0)),
            pl.BlockSpec((N_HID, N_EMB), lambda i: (0, 0)),
            pl.BlockSpec((N_EMB, N_HID), lambda i: (0, 0)),
            pl.BlockSpec((N_HID, 1), lambda i: (0, 0)),
            pl.BlockSpec((N_HID, 1), lambda i: (0, 0)),
        ],
        out_specs=pl.BlockSpec((N_EMB, N_NODES), lambda i: (0, 0)),
        out_shape=jax.ShapeDtypeStruct((N_EMB, N_NODES), f32),
        scratch_shapes=[pltpu.VMEM((N_EMB, ACC_L), f32)],
    )(dist_t, g0, mi3, af_t, wdf_t, wcf_tp, wcf_t, wfc_t, bdf_t, bcf_t)

    return out_t.T
